# baseline (speedup 1.0000x reference)
import functools

import jax
import jax.numpy as jnp
from jax import lax
from jax.experimental import pallas as pl
from jax.experimental.pallas import tpu as pltpu
from jax.experimental.pallas import tpu_sc as plsc

_INFO = plsc.get_sparse_core_info()
_NC = _INFO.num_cores
_NS = _INFO.num_subcores
_NW = _NC * _NS

_B = 16384 * 50
_D = 64
_IW = 128
_B_PER_W = _B // _NW
_NCH = _B_PER_W // _IW
_NBUF = 8
_LOOK = 6
_SLAG = 2

_mesh = plsc.VectorSubcoreMesh(core_axis_name="c", subcore_axis_name="s")


@functools.partial(
    pl.kernel,
    out_type=jax.ShapeDtypeStruct((_B, _D), jnp.float32),
    mesh=_mesh,
    compiler_params=pltpu.CompilerParams(use_tc_tiling_on_sc=False),
    scratch_types=[
        pltpu.VMEM((_NCH, _IW), jnp.int32),
        pltpu.VMEM((_NBUF, _IW, _D), jnp.float32),
    ] + [pltpu.SemaphoreType.DMA] * (2 * _NBUF),
)
def _embed_kernel(idx_hbm, table_hbm, out_hbm, idx_v, rows_v, *sems):
    gsem = sems[:_NBUF]
    ssem = sems[_NBUF:]
    wid = lax.axis_index("s") * _NC + lax.axis_index("c")
    out_base = wid * _B_PER_W

    pltpu.sync_copy(idx_hbm.at[pl.ds(wid * _NCH, _NCH)], idx_v)

    def fire_gather(ch, b):
        pltpu.async_copy(table_hbm.at[idx_v.at[ch]], rows_v.at[b], gsem[b])

    def drain_gather(ch, b):
        pltpu.make_async_copy(
            table_hbm.at[idx_v.at[ch]], rows_v.at[b], gsem[b]).wait()

    def fire_store(ch, b):
        pltpu.async_copy(
            rows_v.at[b], out_hbm.at[pl.ds(out_base + ch * _IW, _IW)],
            ssem[b])

    def drain_store(ch, b):
        pltpu.make_async_copy(
            rows_v.at[b], out_hbm.at[pl.ds(out_base + ch * _IW, _IW)],
            ssem[b]).wait()

    for ch in range(_LOOK):
        fire_gather(ch, ch)

    def body(i, _):
        for bb in range(_NBUF):
            g = i * _NBUF + bb
            drain_gather(g, bb)
            fire_store(g, bb)

            @pl.when(g >= _SLAG)
            def _():
                drain_store(g - _SLAG, (bb - _SLAG) % _NBUF)

            @pl.when(g + _LOOK < _NCH)
            def _():
                fire_gather(g + _LOOK, (bb + _LOOK) % _NBUF)
        return ()

    lax.fori_loop(0, _NCH // _NBUF, body, ())

    for ch in range(_NCH - _SLAG, _NCH):
        drain_store(ch, ch % _NBUF)


def kernel(x, embeddings):
    idx2d = x.reshape(_B // _IW, _IW).astype(jnp.int32)
    out = _embed_kernel(idx2d, embeddings)
    return out.reshape(x.shape[0], x.shape[1], _D)
